# Initial kernel scaffold; baseline (speedup 1.0000x reference)
import jax
import jax.numpy as jnp
from jax.experimental import pallas as pl


def kernel(z, edge_index):
    raise NotImplementedError("write your pallas kernel here")


# speedup vs baseline: 85.5466x; 85.5466x over previous
import functools
import jax
import jax.numpy as jnp
from jax import lax
from jax.experimental import pallas as pl
from jax.experimental.pallas import tpu as pltpu
from jax.experimental.pallas import tpu_sc as plsc

_N_NODES = 100000
_E = 6400000
_NC = 2
_NS = 16
_L = 16
_NW = _NC * _NS
_EPW = _E // _NW
_C1 = 8000
_C2 = 4000
_SLC = 6272
_NPAD = _SLC * _NS

_mesh = plsc.VectorSubcoreMesh(
    core_axis_name="c", subcore_axis_name="s", num_cores=_NC, num_subcores=_NS
)


def _sigmoid(x):
  return 1.0 / (1.0 + jnp.exp(-x))


@functools.partial(
    pl.kernel,
    out_type=jax.ShapeDtypeStruct((_NC, _NPAD), jnp.float32),
    mesh=_mesh,
    compiler_params=pltpu.CompilerParams(needs_layout_passes=False),
    scratch_types=[
        pltpu.VMEM_SHARED((_NPAD,), jnp.float32),
        pltpu.VMEM((_C1,), jnp.int32),
        pltpu.VMEM((_C1,), jnp.float32),
    ],
)
def _deg_kernel(z_hbm, ei_hbm, parts_hbm, deg_sh, row_v, w_v):
  cid = lax.axis_index("c")
  sid = lax.axis_index("s")
  wid = cid * _NS + sid

  def _zero(i, carry):
    w_v[pl.ds(i * _L, _L)] = jnp.zeros((_L,), jnp.float32)
    return carry
  lax.fori_loop(0, _SLC // _L, _zero, 0)
  pltpu.sync_copy(w_v.at[pl.ds(0, _SLC)], deg_sh.at[pl.ds(sid * _SLC, _SLC)])
  plsc.subcore_barrier()

  base = wid * _EPW

  def _chunk(j, carry):
    off = base + j * _C1
    pltpu.sync_copy(ei_hbm.at[pl.ds(off, _C1)], row_v)
    pltpu.sync_copy(z_hbm.at[pl.ds(off, _C1)], w_v)

    def _vec(i, c):
      w_v[pl.ds(i * _L, _L)] = _sigmoid(w_v[pl.ds(i * _L, _L)])
      return c
    lax.fori_loop(0, _C1 // _L, _vec, 0)

    pltpu.sync_copy(w_v, deg_sh.at[row_v], add=True)
    return carry

  lax.fori_loop(0, _EPW // _C1, _chunk, 0)
  plsc.subcore_barrier()

  pltpu.sync_copy(
      deg_sh.at[pl.ds(sid * _SLC, _SLC)],
      parts_hbm.at[cid, pl.ds(sid * _SLC, _SLC)],
  )


@functools.partial(
    pl.kernel,
    out_type=jax.ShapeDtypeStruct((_E,), jnp.float32),
    mesh=_mesh,
    compiler_params=pltpu.CompilerParams(needs_layout_passes=False),
    scratch_types=[
        pltpu.VMEM((_NPAD,), jnp.float32),
        pltpu.VMEM((_SLC,), jnp.float32),
        pltpu.VMEM((_SLC,), jnp.float32),
        pltpu.VMEM((_C2,), jnp.int32),
        pltpu.VMEM((_C2,), jnp.float32),
    ],
)
def _norm_kernel(z_hbm, ei_hbm, parts_hbm, out_hbm, deg_v, t0_v, t1_v, row_v, zo_v):
  cid = lax.axis_index("c")
  sid = lax.axis_index("s")
  wid = cid * _NS + sid

  def _combine(s, carry):
    pltpu.sync_copy(parts_hbm.at[0, pl.ds(s * _SLC, _SLC)], t0_v)
    pltpu.sync_copy(parts_hbm.at[1, pl.ds(s * _SLC, _SLC)], t1_v)

    def _vec(i, c):
      d = t0_v[pl.ds(i * _L, _L)] + t1_v[pl.ds(i * _L, _L)]
      deg_v[pl.ds(s * _SLC + i * _L, _L)] = jnp.maximum(d, 1e-12)
      return c
    lax.fori_loop(0, _SLC // _L, _vec, 0)
    return carry
  lax.fori_loop(0, _NS, _combine, 0)

  base = wid * _EPW

  def _chunk(j, carry):
    off = base + j * _C2
    pltpu.sync_copy(ei_hbm.at[pl.ds(off, _C2)], row_v)
    pltpu.sync_copy(z_hbm.at[pl.ds(off, _C2)], zo_v)

    def _vec(i, c):
      w = _sigmoid(zo_v[pl.ds(i * _L, _L)])
      idx = row_v[pl.ds(i * _L, _L)]
      d = plsc.load_gather(deg_v, [idx])
      zo_v[pl.ds(i * _L, _L)] = w / d
      return c
    lax.fori_loop(0, _C2 // _L, _vec, 0)

    pltpu.sync_copy(zo_v, out_hbm.at[pl.ds(off, _C2)])
    return carry

  lax.fori_loop(0, _EPW // _C2, _chunk, 0)


def kernel(z, edge_index):
  ei_flat = edge_index.reshape(-1)
  parts = _deg_kernel(z, ei_flat)
  return _norm_kernel(z, ei_flat, parts)


# speedup vs baseline: 192.3907x; 2.2490x over previous
import functools
import jax
import jax.numpy as jnp
from jax import lax
from jax.experimental import pallas as pl
from jax.experimental.pallas import tpu as pltpu
from jax.experimental.pallas import tpu_sc as plsc

_N_NODES = 100000
_E = 6400000
_NC = 2
_NS = 16
_L = 16
_NW = _NC * _NS
_EPW = _E // _NW
_C1 = 8000
_NCH1 = _EPW // _C1
_C2 = 4000
_NCH2 = _EPW // _C2
_SLC = 6272
_NPAD = _SLC * _NS
_CSL = 3136
_U = 5

_mesh = plsc.VectorSubcoreMesh(
    core_axis_name="c", subcore_axis_name="s", num_cores=_NC, num_subcores=_NS
)


def _sigmoid(x):
  return 1.0 / (1.0 + jnp.exp(-x))


@functools.partial(
    pl.kernel,
    out_type=(jax.ShapeDtypeStruct((_NPAD,), jnp.float32),
              jax.ShapeDtypeStruct((_NPAD,), jnp.float32)),
    mesh=_mesh,
    compiler_params=pltpu.CompilerParams(needs_layout_passes=False),
    scratch_types=[
        pltpu.VMEM_SHARED((_NPAD,), jnp.float32),
        pltpu.VMEM((_C1,), jnp.int32),
        pltpu.VMEM((_C1,), jnp.int32),
        pltpu.VMEM((_C1,), jnp.int32),
        pltpu.VMEM((_C1,), jnp.float32),
        pltpu.VMEM((_C1,), jnp.float32),
        pltpu.VMEM((_C1,), jnp.float32),
        pltpu.SemaphoreType.DMA,
        pltpu.SemaphoreType.DMA,
        pltpu.SemaphoreType.DMA,
        pltpu.SemaphoreType.DMA,
    ],
)
def _deg_kernel(z_hbm, ei_hbm, p0_hbm, p1_hbm, deg_sh, row0, row1, row2,
                zw0, zw1, zw2, zsem, rsem, ssem0, ssem1):
  cid = lax.axis_index("c")
  sid = lax.axis_index("s")
  wid = cid * _NS + sid
  rows = (row0, row1, row2)
  zws = (zw0, zw1, zw2)
  ssems = (ssem0, ssem1)
  base = wid * _EPW

  def _sigmoid_chunk(zw):
    def _vec(i, carry):
      for k in range(_U):
        s = (i * _U + k) * _L
        zw[pl.ds(s, _L)] = _sigmoid(zw[pl.ds(s, _L)])
      return carry
    lax.fori_loop(0, _C1 // _L // _U, _vec, 0)

  def _pref_in(j, slot):
    pltpu.async_copy(z_hbm.at[pl.ds(base + j * _C1, _C1)], zws[slot], zsem)
    pltpu.async_copy(ei_hbm.at[pl.ds(base + j * _C1, _C1)], rows[slot], rsem)

  def _wait_in(j, slot):
    pltpu.make_async_copy(
        z_hbm.at[pl.ds(base + j * _C1, _C1)], zws[slot], zsem).wait()
    pltpu.make_async_copy(
        ei_hbm.at[pl.ds(base + j * _C1, _C1)], rows[slot], rsem).wait()

  def _scatter(slot, par):
    pltpu.async_copy(zws[slot], deg_sh.at[rows[slot]], ssems[par], add=True)

  def _wait_scatter(slot, par):
    pltpu.make_async_copy(
        zws[slot], deg_sh.at[rows[slot]], ssems[par]).wait()

  def _zero(i, carry):
    zw0[pl.ds(i * _L, _L)] = jnp.zeros((_L,), jnp.float32)
    return carry
  lax.fori_loop(0, _SLC // _L, _zero, 0)
  pltpu.sync_copy(zw0.at[pl.ds(0, _SLC)],
                  deg_sh.at[pl.ds(sid * _SLC, _SLC)])
  plsc.subcore_barrier()

  pltpu.sync_copy(ei_hbm.at[pl.ds(base, _C1)], row0)
  pltpu.sync_copy(z_hbm.at[pl.ds(base, _C1)], zw0)
  _pref_in(1, 1)
  _sigmoid_chunk(zw0)
  _scatter(0, 0)

  def _group(t, carry):
    g = 1 + 6 * t
    for p in range(6):
      jd = g + p
      slot = (1 + p) % 3
      par = (1 + p) % 2
      _wait_in(jd, slot)

      @pl.when(jd >= 2)
      def _():
        _wait_scatter((slot + 1) % 3, par)

      @pl.when(jd + 1 < _NCH1)
      def _():
        _pref_in(jd + 1, (slot + 1) % 3)

      _sigmoid_chunk(zws[slot])
      _scatter(slot, par)
    return carry
  lax.fori_loop(0, (_NCH1 - 1) // 6, _group, 0)

  _wait_scatter((_NCH1 - 2) % 3, (_NCH1 - 2) % 2)
  _wait_scatter((_NCH1 - 1) % 3, (_NCH1 - 1) % 2)
  plsc.subcore_barrier()

  @pl.when(cid == 0)
  def _():
    pltpu.sync_copy(deg_sh.at[pl.ds(sid * _SLC, _SLC)],
                    p0_hbm.at[pl.ds(sid * _SLC, _SLC)])

  @pl.when(cid == 1)
  def _():
    pltpu.sync_copy(deg_sh.at[pl.ds(sid * _SLC, _SLC)],
                    p1_hbm.at[pl.ds(sid * _SLC, _SLC)])


@functools.partial(
    pl.kernel,
    out_type=jax.ShapeDtypeStruct((_E,), jnp.float32),
    mesh=_mesh,
    compiler_params=pltpu.CompilerParams(needs_layout_passes=False),
    scratch_types=[
        pltpu.VMEM((_NPAD,), jnp.float32),
        pltpu.VMEM((_C2,), jnp.int32),
        pltpu.VMEM((_C2,), jnp.int32),
        pltpu.VMEM((_C2,), jnp.int32),
        pltpu.VMEM((_C2,), jnp.float32),
        pltpu.VMEM((_C2,), jnp.float32),
        pltpu.VMEM((_C2,), jnp.float32),
        pltpu.SemaphoreType.DMA,
        pltpu.SemaphoreType.DMA,
        pltpu.SemaphoreType.DMA,
        pltpu.SemaphoreType.DMA,
    ],
)
def _norm_kernel(z_hbm, ei_hbm, p0_hbm, p1_hbm, out_hbm, deg_v, row0, row1,
                 row2, zo0, zo1, zo2, zsem, rsem, osem0, osem1):
  cid = lax.axis_index("c")
  sid = lax.axis_index("s")
  wid = cid * _NS + sid
  rows = (row0, row1, row2)
  zos = (zo0, zo1, zo2)
  osems = (osem0, osem1)
  base = wid * _EPW

  def _combine(s, carry):
    pltpu.sync_copy(p0_hbm.at[pl.ds(s * _CSL, _CSL)],
                    zo0.at[pl.ds(0, _CSL)])
    pltpu.sync_copy(p1_hbm.at[pl.ds(s * _CSL, _CSL)],
                    zo1.at[pl.ds(0, _CSL)])

    def _vec(i, c):
      d = zo0[pl.ds(i * _L, _L)] + zo1[pl.ds(i * _L, _L)]
      deg_v[pl.ds(s * _CSL + i * _L, _L)] = jnp.maximum(d, 1e-12)
      return c
    lax.fori_loop(0, _CSL // _L, _vec, 0)
    return carry
  lax.fori_loop(0, _NPAD // _CSL, _combine, 0)

  def _norm_chunk(zo, row):
    def _vec(i, carry):
      for k in range(_U):
        s = (i * _U + k) * _L
        w = _sigmoid(zo[pl.ds(s, _L)])
        idx = row[pl.ds(s, _L)]
        d = plsc.load_gather(deg_v, [idx])
        zo[pl.ds(s, _L)] = w / d
      return carry
    lax.fori_loop(0, _C2 // _L // _U, _vec, 0)

  def _pref_in(j, slot):
    pltpu.async_copy(z_hbm.at[pl.ds(base + j * _C2, _C2)], zos[slot], zsem)
    pltpu.async_copy(ei_hbm.at[pl.ds(base + j * _C2, _C2)], rows[slot], rsem)

  def _wait_in(j, slot):
    pltpu.make_async_copy(
        z_hbm.at[pl.ds(base + j * _C2, _C2)], zos[slot], zsem).wait()
    pltpu.make_async_copy(
        ei_hbm.at[pl.ds(base + j * _C2, _C2)], rows[slot], rsem).wait()

  def _out(j, slot, par):
    pltpu.async_copy(zos[slot], out_hbm.at[pl.ds(base + j * _C2, _C2)],
                     osems[par])

  def _wait_out(j, slot, par):
    pltpu.make_async_copy(
        zos[slot], out_hbm.at[pl.ds(base + j * _C2, _C2)], osems[par]).wait()

  pltpu.sync_copy(ei_hbm.at[pl.ds(base, _C2)], row0)
  pltpu.sync_copy(z_hbm.at[pl.ds(base, _C2)], zo0)
  pltpu.sync_copy(ei_hbm.at[pl.ds(base + _C2, _C2)], row1)
  pltpu.sync_copy(z_hbm.at[pl.ds(base + _C2, _C2)], zo1)
  _pref_in(2, 2)
  _norm_chunk(zo0, row0)
  _out(0, 0, 0)
  _norm_chunk(zo1, row1)
  _out(1, 1, 1)

  def _group(t, carry):
    g = 2 + 6 * t
    for p in range(6):
      jd = g + p
      slot = (2 + p) % 3
      par = p % 2
      _wait_in(jd, slot)
      _wait_out(jd - 2, (slot + 1) % 3, par)

      @pl.when(jd + 1 < _NCH2)
      def _():
        _pref_in(jd + 1, (slot + 1) % 3)

      _norm_chunk(zos[slot], rows[slot])
      _out(jd, slot, par)
    return carry
  lax.fori_loop(0, (_NCH2 - 2) // 6, _group, 0)

  _wait_out(_NCH2 - 2, (_NCH2 - 2) % 3, (_NCH2 - 2) % 2)
  _wait_out(_NCH2 - 1, (_NCH2 - 1) % 3, (_NCH2 - 1) % 2)


def kernel(z, edge_index):
  ei_flat = edge_index.reshape(-1)
  p0, p1 = _deg_kernel(z, ei_flat)
  return _norm_kernel(z, ei_flat, p0, p1)


# speedup vs baseline: 292.2273x; 1.5189x over previous
import functools
import jax
import jax.numpy as jnp
from jax import lax
from jax.experimental import pallas as pl
from jax.experimental.pallas import tpu as pltpu
from jax.experimental.pallas import tpu_sc as plsc

_N_NODES = 100000
_E = 6400000
_NC = 2
_NS = 16
_L = 16
_NW = _NC * _NS
_EPW = _E // _NW
_C1 = 8000
_NCH1 = _EPW // _C1
_C2 = 4000
_NCH2 = _EPW // _C2
_SLC = 6272
_NPAD = _SLC * _NS
_CSL = 3136
_U = 5

_mesh = plsc.VectorSubcoreMesh(
    core_axis_name="c", subcore_axis_name="s", num_cores=_NC, num_subcores=_NS
)


def _sigmoid(x):
  return 1.0 / (1.0 + jnp.exp(-x))


@functools.partial(
    pl.kernel,
    out_type=(jax.ShapeDtypeStruct((_NPAD,), jnp.float32),
              jax.ShapeDtypeStruct((_NPAD,), jnp.float32)),
    mesh=_mesh,
    compiler_params=pltpu.CompilerParams(needs_layout_passes=False),
    scratch_types=[
        pltpu.VMEM_SHARED((_NPAD,), jnp.float32),
        pltpu.VMEM((_C1,), jnp.int32),
        pltpu.VMEM((_C1,), jnp.int32),
        pltpu.VMEM((_C1,), jnp.int32),
        pltpu.VMEM((_C1,), jnp.float32),
        pltpu.VMEM((_C1,), jnp.float32),
        pltpu.VMEM((_C1,), jnp.float32),
        pltpu.SemaphoreType.DMA,
        pltpu.SemaphoreType.DMA,
        pltpu.SemaphoreType.DMA,
        pltpu.SemaphoreType.DMA,
    ],
)
def _deg_kernel(z_hbm, ei_hbm, p0_hbm, p1_hbm, deg_sh, row0, row1, row2,
                zw0, zw1, zw2, zsem, rsem, ssem0, ssem1):
  cid = lax.axis_index("c")
  sid = lax.axis_index("s")
  wid = cid * _NS + sid
  rows = (row0, row1, row2)
  zws = (zw0, zw1, zw2)
  ssems = (ssem0, ssem1)
  base = wid * _EPW

  def _sigmoid_chunk(zw):
    @plsc.parallel_loop(0, _C1, _L, unroll=_U)
    def _vec(s):
      zw[pl.ds(s, _L)] = _sigmoid(zw[pl.ds(s, _L)])

  def _pref_in(j, slot):
    pltpu.async_copy(z_hbm.at[pl.ds(base + j * _C1, _C1)], zws[slot], zsem)
    pltpu.async_copy(ei_hbm.at[pl.ds(base + j * _C1, _C1)], rows[slot], rsem)

  def _wait_in(j, slot):
    pltpu.make_async_copy(
        z_hbm.at[pl.ds(base + j * _C1, _C1)], zws[slot], zsem).wait()
    pltpu.make_async_copy(
        ei_hbm.at[pl.ds(base + j * _C1, _C1)], rows[slot], rsem).wait()

  def _scatter(slot, par):
    pltpu.async_copy(zws[slot], deg_sh.at[rows[slot]], ssems[par], add=True)

  def _wait_scatter(slot, par):
    pltpu.make_async_copy(
        zws[slot], deg_sh.at[rows[slot]], ssems[par]).wait()

  @plsc.parallel_loop(0, _SLC, _L, unroll=8)
  def _zero(s):
    zw0[pl.ds(s, _L)] = jnp.zeros((_L,), jnp.float32)
  pltpu.sync_copy(zw0.at[pl.ds(0, _SLC)],
                  deg_sh.at[pl.ds(sid * _SLC, _SLC)])
  plsc.subcore_barrier()

  pltpu.sync_copy(ei_hbm.at[pl.ds(base, _C1)], row0)
  pltpu.sync_copy(z_hbm.at[pl.ds(base, _C1)], zw0)
  _pref_in(1, 1)
  _sigmoid_chunk(zw0)
  _scatter(0, 0)

  def _group(t, carry):
    g = 1 + 6 * t
    for p in range(6):
      jd = g + p
      slot = (1 + p) % 3
      par = (1 + p) % 2
      _wait_in(jd, slot)

      @pl.when(jd >= 2)
      def _():
        _wait_scatter((slot + 1) % 3, par)

      @pl.when(jd + 1 < _NCH1)
      def _():
        _pref_in(jd + 1, (slot + 1) % 3)

      _sigmoid_chunk(zws[slot])
      _scatter(slot, par)
    return carry
  lax.fori_loop(0, (_NCH1 - 1) // 6, _group, 0)

  _wait_scatter((_NCH1 - 2) % 3, (_NCH1 - 2) % 2)
  _wait_scatter((_NCH1 - 1) % 3, (_NCH1 - 1) % 2)
  plsc.subcore_barrier()

  @pl.when(cid == 0)
  def _():
    pltpu.sync_copy(deg_sh.at[pl.ds(sid * _SLC, _SLC)],
                    p0_hbm.at[pl.ds(sid * _SLC, _SLC)])

  @pl.when(cid == 1)
  def _():
    pltpu.sync_copy(deg_sh.at[pl.ds(sid * _SLC, _SLC)],
                    p1_hbm.at[pl.ds(sid * _SLC, _SLC)])


@functools.partial(
    pl.kernel,
    out_type=jax.ShapeDtypeStruct((_E,), jnp.float32),
    mesh=_mesh,
    compiler_params=pltpu.CompilerParams(needs_layout_passes=False),
    scratch_types=[
        pltpu.VMEM((_NPAD,), jnp.float32),
        pltpu.VMEM((_C2,), jnp.int32),
        pltpu.VMEM((_C2,), jnp.int32),
        pltpu.VMEM((_C2,), jnp.int32),
        pltpu.VMEM((_C2,), jnp.float32),
        pltpu.VMEM((_C2,), jnp.float32),
        pltpu.VMEM((_C2,), jnp.float32),
        pltpu.SemaphoreType.DMA,
        pltpu.SemaphoreType.DMA,
        pltpu.SemaphoreType.DMA,
        pltpu.SemaphoreType.DMA,
    ],
)
def _norm_kernel(z_hbm, ei_hbm, p0_hbm, p1_hbm, out_hbm, deg_v, row0, row1,
                 row2, zo0, zo1, zo2, zsem, rsem, osem0, osem1):
  cid = lax.axis_index("c")
  sid = lax.axis_index("s")
  wid = cid * _NS + sid
  rows = (row0, row1, row2)
  zos = (zo0, zo1, zo2)
  osems = (osem0, osem1)
  base = wid * _EPW

  def _combine(s, carry):
    pltpu.sync_copy(p0_hbm.at[pl.ds(s * _CSL, _CSL)],
                    zo0.at[pl.ds(0, _CSL)])
    pltpu.sync_copy(p1_hbm.at[pl.ds(s * _CSL, _CSL)],
                    zo1.at[pl.ds(0, _CSL)])

    @plsc.parallel_loop(0, _CSL, _L, unroll=7)
    def _vec(i):
      d = zo0[pl.ds(i, _L)] + zo1[pl.ds(i, _L)]
      deg_v[pl.ds(s * _CSL + i, _L)] = jnp.maximum(d, 1e-12)
    return carry
  lax.fori_loop(0, _NPAD // _CSL, _combine, 0)

  def _norm_chunk(zo, row):
    @plsc.parallel_loop(0, _C2, _L, unroll=_U)
    def _vec(s):
      den = 1.0 + jnp.exp(-zo[pl.ds(s, _L)])
      idx = row[pl.ds(s, _L)]
      d = plsc.load_gather(deg_v, [idx])
      zo[pl.ds(s, _L)] = 1.0 / (den * d)

  def _pref_in(j, slot):
    pltpu.async_copy(z_hbm.at[pl.ds(base + j * _C2, _C2)], zos[slot], zsem)
    pltpu.async_copy(ei_hbm.at[pl.ds(base + j * _C2, _C2)], rows[slot], rsem)

  def _wait_in(j, slot):
    pltpu.make_async_copy(
        z_hbm.at[pl.ds(base + j * _C2, _C2)], zos[slot], zsem).wait()
    pltpu.make_async_copy(
        ei_hbm.at[pl.ds(base + j * _C2, _C2)], rows[slot], rsem).wait()

  def _out(j, slot, par):
    pltpu.async_copy(zos[slot], out_hbm.at[pl.ds(base + j * _C2, _C2)],
                     osems[par])

  def _wait_out(j, slot, par):
    pltpu.make_async_copy(
        zos[slot], out_hbm.at[pl.ds(base + j * _C2, _C2)], osems[par]).wait()

  pltpu.sync_copy(ei_hbm.at[pl.ds(base, _C2)], row0)
  pltpu.sync_copy(z_hbm.at[pl.ds(base, _C2)], zo0)
  pltpu.sync_copy(ei_hbm.at[pl.ds(base + _C2, _C2)], row1)
  pltpu.sync_copy(z_hbm.at[pl.ds(base + _C2, _C2)], zo1)
  _pref_in(2, 2)
  _norm_chunk(zo0, row0)
  _out(0, 0, 0)
  _norm_chunk(zo1, row1)
  _out(1, 1, 1)

  def _group(t, carry):
    g = 2 + 6 * t
    for p in range(6):
      jd = g + p
      slot = (2 + p) % 3
      par = p % 2
      _wait_in(jd, slot)
      _wait_out(jd - 2, (slot + 1) % 3, par)

      @pl.when(jd + 1 < _NCH2)
      def _():
        _pref_in(jd + 1, (slot + 1) % 3)

      _norm_chunk(zos[slot], rows[slot])
      _out(jd, slot, par)
    return carry
  lax.fori_loop(0, (_NCH2 - 2) // 6, _group, 0)

  _wait_out(_NCH2 - 2, (_NCH2 - 2) % 3, (_NCH2 - 2) % 2)
  _wait_out(_NCH2 - 1, (_NCH2 - 1) % 3, (_NCH2 - 1) % 2)


def kernel(z, edge_index):
  ei_flat = edge_index.reshape(-1)
  p0, p1 = _deg_kernel(z, ei_flat)
  return _norm_kernel(z, ei_flat, p0, p1)


# speedup vs baseline: 320.9298x; 1.0982x over previous
import functools
import jax
import jax.numpy as jnp
from jax import lax
from jax.experimental import pallas as pl
from jax.experimental.pallas import tpu as pltpu
from jax.experimental.pallas import tpu_sc as plsc

_N_NODES = 100000
_E = 6400000
_NC = 2
_NS = 16
_L = 16
_NW = _NC * _NS
_EPW = _E // _NW
_C1 = 8000
_NCH1 = _EPW // _C1
_C2 = 4000
_NCH2 = _EPW // _C2
_SLC = 6272
_NPAD = _SLC * _NS
_CSL = 3136
_U = 10

_mesh = plsc.VectorSubcoreMesh(
    core_axis_name="c", subcore_axis_name="s", num_cores=_NC, num_subcores=_NS
)


def _sigmoid(x):
  return 1.0 / (1.0 + jnp.exp(-x))


@functools.partial(
    pl.kernel,
    out_type=(jax.ShapeDtypeStruct((_NPAD,), jnp.float32),
              jax.ShapeDtypeStruct((_NPAD,), jnp.float32)),
    mesh=_mesh,
    compiler_params=pltpu.CompilerParams(needs_layout_passes=False),
    scratch_types=[
        pltpu.VMEM_SHARED((_NPAD,), jnp.float32),
        pltpu.VMEM((_C1,), jnp.int32),
        pltpu.VMEM((_C1,), jnp.int32),
        pltpu.VMEM((_C1,), jnp.int32),
        pltpu.VMEM((_C1,), jnp.float32),
        pltpu.VMEM((_C1,), jnp.float32),
        pltpu.VMEM((_C1,), jnp.float32),
        pltpu.SemaphoreType.DMA,
        pltpu.SemaphoreType.DMA,
        pltpu.SemaphoreType.DMA,
        pltpu.SemaphoreType.DMA,
    ],
)
def _deg_kernel(z_hbm, ei_hbm, p0_hbm, p1_hbm, deg_sh, row0, row1, row2,
                zw0, zw1, zw2, zsem, rsem, ssem0, ssem1):
  cid = lax.axis_index("c")
  sid = lax.axis_index("s")
  wid = cid * _NS + sid
  rows = (row0, row1, row2)
  zws = (zw0, zw1, zw2)
  ssems = (ssem0, ssem1)
  base = wid * _EPW

  def _sigmoid_chunk(zw):
    @plsc.parallel_loop(0, _C1, _L, unroll=_U)
    def _vec(s):
      zw[pl.ds(s, _L)] = _sigmoid(zw[pl.ds(s, _L)])

  def _pref_in(j, slot):
    pltpu.async_copy(z_hbm.at[pl.ds(base + j * _C1, _C1)], zws[slot], zsem)
    pltpu.async_copy(ei_hbm.at[pl.ds(base + j * _C1, _C1)], rows[slot], rsem)

  def _wait_in(j, slot):
    pltpu.make_async_copy(
        z_hbm.at[pl.ds(base + j * _C1, _C1)], zws[slot], zsem).wait()
    pltpu.make_async_copy(
        ei_hbm.at[pl.ds(base + j * _C1, _C1)], rows[slot], rsem).wait()

  def _scatter(slot, par):
    pltpu.async_copy(zws[slot], deg_sh.at[rows[slot]], ssems[par], add=True)

  def _wait_scatter(slot, par):
    pltpu.make_async_copy(
        zws[slot], deg_sh.at[rows[slot]], ssems[par]).wait()

  @plsc.parallel_loop(0, _SLC, _L, unroll=8)
  def _zero(s):
    zw0[pl.ds(s, _L)] = jnp.zeros((_L,), jnp.float32)
  pltpu.sync_copy(zw0.at[pl.ds(0, _SLC)],
                  deg_sh.at[pl.ds(sid * _SLC, _SLC)])
  plsc.subcore_barrier()

  pltpu.sync_copy(ei_hbm.at[pl.ds(base, _C1)], row0)
  pltpu.sync_copy(z_hbm.at[pl.ds(base, _C1)], zw0)
  _pref_in(1, 1)
  _sigmoid_chunk(zw0)
  _scatter(0, 0)

  def _group(t, carry):
    g = 1 + 6 * t
    for p in range(6):
      jd = g + p
      slot = (1 + p) % 3
      par = (1 + p) % 2
      _wait_in(jd, slot)

      @pl.when(jd >= 2)
      def _():
        _wait_scatter((slot + 1) % 3, par)

      @pl.when(jd + 1 < _NCH1)
      def _():
        _pref_in(jd + 1, (slot + 1) % 3)

      _sigmoid_chunk(zws[slot])
      _scatter(slot, par)
    return carry
  lax.fori_loop(0, (_NCH1 - 1) // 6, _group, 0)

  _wait_scatter((_NCH1 - 2) % 3, (_NCH1 - 2) % 2)
  _wait_scatter((_NCH1 - 1) % 3, (_NCH1 - 1) % 2)
  plsc.subcore_barrier()

  @pl.when(cid == 0)
  def _():
    pltpu.sync_copy(deg_sh.at[pl.ds(sid * _SLC, _SLC)],
                    p0_hbm.at[pl.ds(sid * _SLC, _SLC)])

  @pl.when(cid == 1)
  def _():
    pltpu.sync_copy(deg_sh.at[pl.ds(sid * _SLC, _SLC)],
                    p1_hbm.at[pl.ds(sid * _SLC, _SLC)])


@functools.partial(
    pl.kernel,
    out_type=jax.ShapeDtypeStruct((_E,), jnp.float32),
    mesh=_mesh,
    compiler_params=pltpu.CompilerParams(needs_layout_passes=False),
    scratch_types=[
        pltpu.VMEM((_NPAD,), jnp.float32),
        pltpu.VMEM((_C2,), jnp.int32),
        pltpu.VMEM((_C2,), jnp.int32),
        pltpu.VMEM((_C2,), jnp.int32),
        pltpu.VMEM((_C2,), jnp.float32),
        pltpu.VMEM((_C2,), jnp.float32),
        pltpu.VMEM((_C2,), jnp.float32),
        pltpu.SemaphoreType.DMA,
        pltpu.SemaphoreType.DMA,
        pltpu.SemaphoreType.DMA,
        pltpu.SemaphoreType.DMA,
    ],
)
def _norm_kernel(z_hbm, ei_hbm, p0_hbm, p1_hbm, out_hbm, deg_v, row0, row1,
                 row2, zo0, zo1, zo2, zsem, rsem, osem0, osem1):
  cid = lax.axis_index("c")
  sid = lax.axis_index("s")
  wid = cid * _NS + sid
  rows = (row0, row1, row2)
  zos = (zo0, zo1, zo2)
  osems = (osem0, osem1)
  base = wid * _EPW

  pltpu.sync_copy(p0_hbm, deg_v)
  zbufs = (zo0, zo1)
  pltpu.async_copy(p1_hbm.at[pl.ds(0, _CSL)], zo0.at[pl.ds(0, _CSL)], zsem)

  def _combine(t, carry):
    for p in range(2):
      s = 2 * t + p
      buf = zbufs[p]
      sem = (zsem, rsem)[p]
      pltpu.make_async_copy(p1_hbm.at[pl.ds(s * _CSL, _CSL)],
                            buf.at[pl.ds(0, _CSL)], sem).wait()

      @pl.when(s + 1 < _NPAD // _CSL)
      def _():
        pltpu.async_copy(p1_hbm.at[pl.ds((s + 1) * _CSL, _CSL)],
                         zbufs[1 - p].at[pl.ds(0, _CSL)], (zsem, rsem)[1 - p])

      @plsc.parallel_loop(0, _CSL, _L, unroll=7)
      def _vec(i):
        d = deg_v[pl.ds(s * _CSL + i, _L)] + buf[pl.ds(i, _L)]
        deg_v[pl.ds(s * _CSL + i, _L)] = jnp.maximum(d, 1e-12)
    return carry
  lax.fori_loop(0, _NPAD // _CSL // 2, _combine, 0)

  def _norm_chunk(zo, row):
    @plsc.parallel_loop(0, _C2, _L, unroll=_U)
    def _vec(s):
      den = 1.0 + jnp.exp(-zo[pl.ds(s, _L)])
      idx = row[pl.ds(s, _L)]
      d = plsc.load_gather(deg_v, [idx])
      zo[pl.ds(s, _L)] = 1.0 / (den * d)

  def _pref_in(j, slot):
    pltpu.async_copy(z_hbm.at[pl.ds(base + j * _C2, _C2)], zos[slot], zsem)
    pltpu.async_copy(ei_hbm.at[pl.ds(base + j * _C2, _C2)], rows[slot], rsem)

  def _wait_in(j, slot):
    pltpu.make_async_copy(
        z_hbm.at[pl.ds(base + j * _C2, _C2)], zos[slot], zsem).wait()
    pltpu.make_async_copy(
        ei_hbm.at[pl.ds(base + j * _C2, _C2)], rows[slot], rsem).wait()

  def _out(j, slot, par):
    pltpu.async_copy(zos[slot], out_hbm.at[pl.ds(base + j * _C2, _C2)],
                     osems[par])

  def _wait_out(j, slot, par):
    pltpu.make_async_copy(
        zos[slot], out_hbm.at[pl.ds(base + j * _C2, _C2)], osems[par]).wait()

  pltpu.sync_copy(ei_hbm.at[pl.ds(base, _C2)], row0)
  pltpu.sync_copy(z_hbm.at[pl.ds(base, _C2)], zo0)
  pltpu.sync_copy(ei_hbm.at[pl.ds(base + _C2, _C2)], row1)
  pltpu.sync_copy(z_hbm.at[pl.ds(base + _C2, _C2)], zo1)
  _pref_in(2, 2)
  _norm_chunk(zo0, row0)
  _out(0, 0, 0)
  _norm_chunk(zo1, row1)
  _out(1, 1, 1)

  def _group(t, carry):
    g = 2 + 6 * t
    for p in range(6):
      jd = g + p
      slot = (2 + p) % 3
      par = p % 2
      _wait_in(jd, slot)
      _wait_out(jd - 2, (slot + 1) % 3, par)

      @pl.when(jd + 1 < _NCH2)
      def _():
        _pref_in(jd + 1, (slot + 1) % 3)

      _norm_chunk(zos[slot], rows[slot])
      _out(jd, slot, par)
    return carry
  lax.fori_loop(0, (_NCH2 - 2) // 6, _group, 0)

  _wait_out(_NCH2 - 2, (_NCH2 - 2) % 3, (_NCH2 - 2) % 2)
  _wait_out(_NCH2 - 1, (_NCH2 - 1) % 3, (_NCH2 - 1) % 2)


def kernel(z, edge_index):
  ei_flat = edge_index.reshape(-1)
  p0, p1 = _deg_kernel(z, ei_flat)
  return _norm_kernel(z, ei_flat, p0, p1)
